# Initial kernel scaffold; baseline (speedup 1.0000x reference)
import jax
import jax.numpy as jnp
from jax.experimental import pallas as pl


def kernel(s, v, p, edge_index_global, edge_attr_global, batch, params):
    raise NotImplementedError("write your pallas kernel here")


# speedup vs baseline: 4.0619x; 4.0619x over previous
import functools
import jax
import jax.numpy as jnp
import numpy as np
from jax.experimental import pallas as pl
from jax.experimental.pallas import tpu as pltpu

N = 1024
EG = 32768
SDIM = 256
VDIM = 64
EDIM = 16
K = 32
CUT = 5.0
NCONV = 3

_F32 = jnp.float32


def _col(x3, c):
    sel = (jax.lax.broadcasted_iota(jnp.int32, (1, 3), 1) == c).astype(_F32)
    return jnp.sum(x3 * sel, axis=1, keepdims=True)


def _mid_body(src_ref, valid_ref, eattr_ref, A_ref, p_ref, pn_ref,
              vx_ref, vy_ref, vz_ref, B_ref, pd_ref, pnd_ref,
              w1c_ref, wda_ref, w2p_ref, b2p_ref, w2g_ref, b2g_ref,
              outs_ref, outv_ref, outp_ref, *, D, E):
    src = src_ref[...]
    n_iota = jax.lax.broadcasted_iota(jnp.int32, (E, N), 1)
    oh = (src == n_iota).astype(_F32)
    Asrc = jnp.dot(oh, A_ref[...], preferred_element_type=_F32)
    psrc = jnp.dot(oh, p_ref[...], preferred_element_type=_F32)
    pnsrc = jnp.dot(oh, pn_ref[...], preferred_element_type=_F32)
    vxs = jnp.dot(oh, vx_ref[...], preferred_element_type=_F32)
    vys = jnp.dot(oh, vy_ref[...], preferred_element_type=_F32)
    vzs = jnp.dot(oh, vz_ref[...], preferred_element_type=_F32)

    di = jax.lax.broadcasted_iota(jnp.int32, (E, D), 1)
    ei = jax.lax.broadcasted_iota(jnp.int32, (E, D), 0) // K
    ohd = (di == ei).astype(_F32)
    Bdst = jnp.dot(ohd, B_ref[...], preferred_element_type=_F32)
    pdst = jnp.dot(ohd, pd_ref[...], preferred_element_type=_F32)
    pndst = jnp.dot(ohd, pnd_ref[...], preferred_element_type=_F32)

    r = pdst - psrc
    d2 = jnp.clip(jnp.sum(r * r, axis=1, keepdims=True), 1e-6, None)
    d = jnp.sqrt(d2)
    a = jnp.sum(pndst * pnsrc, axis=1, keepdims=True)
    rn = r / (1.0 + d)

    eproj = jnp.dot(eattr_ref[...], w1c_ref[...], preferred_element_type=_F32)
    wd = wda_ref[0:1, :]
    wa = wda_ref[1:2, :]
    h = Asrc + Bdst + eproj + d * wd + a * wa
    hs = h * jax.nn.sigmoid(h)
    o = jnp.dot(hs, w2p_ref[...], preferred_element_type=_F32) + b2p_ref[...]
    m_s = o[:, 0:SDIM]
    gvv = o[:, SDIM:SDIM + 128]
    gvr = o[:, SDIM + 128:SDIM + 256]
    gp = jnp.dot(hs, w2g_ref[...], preferred_element_type=_F32) + b2g_ref[...]

    env = 0.5 * (jnp.cos(jnp.pi * jnp.minimum(d, CUT) / CUT) + 1.0)
    env = env * (d < CUT).astype(_F32)
    w = env * valid_ref[...]

    mvx = gvr * _col(rn, 0) + gvv * vxs
    mvy = gvr * _col(rn, 1) + gvv * vys
    mvz = gvr * _col(rn, 2) + gvv * vzs

    dd = jax.lax.broadcasted_iota(jnp.int32, (D, E), 0)
    de = jax.lax.broadcasted_iota(jnp.int32, (D, E), 1) // K
    ind = (dd == de).astype(_F32)
    aggs = jnp.dot(ind, m_s * w, preferred_element_type=_F32)
    cnt = jnp.dot(ind, w, preferred_element_type=_F32) + 1e-6
    outs_ref[...] = aggs
    outv_ref[:, 0:128] = jnp.dot(ind, mvx * w, preferred_element_type=_F32) / cnt
    outv_ref[:, 128:256] = jnp.dot(ind, mvy * w, preferred_element_type=_F32) / cnt
    outv_ref[:, 256:384] = jnp.dot(ind, mvz * w, preferred_element_type=_F32) / cnt
    outp_ref[...] = jnp.dot(ind, rn * (gp * w), preferred_element_type=_F32) / cnt


def _mid_conv_aggregate(src2d, valid2d, eattr, A, B, p, pn, vpad, wpack, D=32):
    E = D * K
    grid = N // D
    full = lambda shape: pl.BlockSpec(shape, lambda g: (0, 0))
    blk = lambda shape: pl.BlockSpec(shape, lambda g: (g, 0))
    w1c, wda, w2p, b2p, w2g, b2g = wpack
    return pl.pallas_call(
        functools.partial(_mid_body, D=D, E=E),
        grid=(grid,),
        in_specs=[
            blk((E, 1)), blk((E, 1)), blk((E, EDIM)),
            full((N, SDIM)), full((N, 3)), full((N, 3)),
            full((N, 128)), full((N, 128)), full((N, 128)),
            blk((D, SDIM)), blk((D, 3)), blk((D, 3)),
            full((EDIM, SDIM)), full((2, SDIM)), full((SDIM, 512)),
            full((1, 512)), full((SDIM, 1)), full((1, 1)),
        ],
        out_specs=[blk((D, SDIM)), blk((D, 384)), blk((D, 3))],
        out_shape=[
            jax.ShapeDtypeStruct((N, SDIM), _F32),
            jax.ShapeDtypeStruct((N, 384), _F32),
            jax.ShapeDtypeStruct((N, 3), _F32),
        ],
    )(src2d, valid2d, eattr, A, p, pn, vpad[0], vpad[1], vpad[2],
      B, p, pn, w1c, wda, w2p, b2p, w2g, b2g)


def _pp_body(src_ref, dst_ref, eattr_ref, A_ref, p_ref, pn_ref, B_ref,
             w1c_ref, wda_ref, w2p_ref, b2p_ref, w2g_ref, b2g_ref,
             outs_ref, outv_ref, outp_ref, outc_ref, *, E):
    @pl.when(pl.program_id(0) == 0)
    def _init():
        outs_ref[...] = jnp.zeros_like(outs_ref)
        outv_ref[...] = jnp.zeros_like(outv_ref)
        outp_ref[...] = jnp.zeros_like(outp_ref)
        outc_ref[...] = jnp.zeros_like(outc_ref)

    n_iota = jax.lax.broadcasted_iota(jnp.int32, (E, N), 1)
    oh = (src_ref[...] == n_iota).astype(_F32)
    ohd = (dst_ref[...] == n_iota).astype(_F32)
    Asrc = jnp.dot(oh, A_ref[...], preferred_element_type=_F32)
    psrc = jnp.dot(oh, p_ref[...], preferred_element_type=_F32)
    pnsrc = jnp.dot(oh, pn_ref[...], preferred_element_type=_F32)
    Bdst = jnp.dot(ohd, B_ref[...], preferred_element_type=_F32)
    pdst = jnp.dot(ohd, p_ref[...], preferred_element_type=_F32)
    pndst = jnp.dot(ohd, pn_ref[...], preferred_element_type=_F32)

    r = pdst - psrc
    d2 = jnp.clip(jnp.sum(r * r, axis=1, keepdims=True), 1e-6, None)
    d = jnp.sqrt(d2)
    a = jnp.sum(pndst * pnsrc, axis=1, keepdims=True)
    rn = r / (1.0 + d)

    eproj = jnp.dot(eattr_ref[...], w1c_ref[...], preferred_element_type=_F32)
    h = Asrc + Bdst + eproj + d * wda_ref[0:1, :] + a * wda_ref[1:2, :]
    hs = h * jax.nn.sigmoid(h)
    o = jnp.dot(hs, w2p_ref[...], preferred_element_type=_F32) + b2p_ref[...]
    m_s = o[:, 0:SDIM]
    gvr = o[:, SDIM + 128:SDIM + 256]
    gp = jnp.dot(hs, w2g_ref[...], preferred_element_type=_F32) + b2g_ref[...]

    scat = lambda x: jax.lax.dot_general(
        ohd, x, (((0,), (0,)), ((), ())), preferred_element_type=_F32)
    outs_ref[...] += scat(m_s)
    outv_ref[:, 0:128] += scat(gvr * _col(rn, 0))
    outv_ref[:, 128:256] += scat(gvr * _col(rn, 1))
    outv_ref[:, 256:384] += scat(gvr * _col(rn, 2))
    outp_ref[...] += scat(rn * gp)
    outc_ref[...] += scat(jnp.ones_like(gp))


def _pp_conv_aggregate(src2d, dst2d, eattr, A, B, p, pn, wpack, E=1024):
    grid = EG // E
    full = lambda shape: pl.BlockSpec(shape, lambda g: (0, 0))
    blk = lambda shape: pl.BlockSpec(shape, lambda g: (g, 0))
    acc = lambda shape: pl.BlockSpec(shape, lambda g: (0, 0))
    w1c, wda, w2p, b2p, w2g, b2g = wpack
    return pl.pallas_call(
        functools.partial(_pp_body, E=E),
        grid=(grid,),
        in_specs=[
            blk((E, 1)), blk((E, 1)), blk((E, EDIM)),
            full((N, SDIM)), full((N, 3)), full((N, 3)), full((N, SDIM)),
            full((EDIM, SDIM)), full((2, SDIM)), full((SDIM, 512)),
            full((1, 512)), full((SDIM, 1)), full((1, 1)),
        ],
        out_specs=[acc((N, SDIM)), acc((N, 384)), acc((N, 3)), acc((N, 1))],
        out_shape=[
            jax.ShapeDtypeStruct((N, SDIM), _F32),
            jax.ShapeDtypeStruct((N, 384), _F32),
            jax.ShapeDtypeStruct((N, 3), _F32),
            jax.ShapeDtypeStruct((N, 1), _F32),
        ],
    )(src2d, dst2d, eattr, A, p, pn, B, w1c, wda, w2p, b2p, w2g, b2g)


def _prep_conv_weights(cp):
    W1 = cp["m1"]["W"]
    b1 = cp["m1"]["b"]
    W1a = W1[0:SDIM]
    W1b = W1[SDIM:2 * SDIM]
    w1c = W1[2 * SDIM:2 * SDIM + EDIM]
    wda = W1[2 * SDIM + EDIM:2 * SDIM + EDIM + 2]
    W2 = cp["m2"]["W"]
    b2 = cp["m2"]["b"]
    w2p = jnp.zeros((SDIM, 512), _F32)
    w2p = w2p.at[:, 0:SDIM].set(W2[:, 0:SDIM])
    w2p = w2p.at[:, SDIM:SDIM + VDIM].set(W2[:, SDIM:SDIM + VDIM])
    w2p = w2p.at[:, SDIM + 128:SDIM + 128 + VDIM].set(W2[:, SDIM + VDIM:SDIM + 2 * VDIM])
    b2p = jnp.zeros((1, 512), _F32)
    b2p = b2p.at[0, 0:SDIM].set(b2[0:SDIM])
    b2p = b2p.at[0, SDIM:SDIM + VDIM].set(b2[SDIM:SDIM + VDIM])
    b2p = b2p.at[0, SDIM + 128:SDIM + 128 + VDIM].set(b2[SDIM + VDIM:SDIM + 2 * VDIM])
    w2g = W2[:, -1:]
    b2g = b2[-1:].reshape(1, 1)
    return W1a, W1b, b1, (w1c, wda, w2p, b2p, w2g, b2g)


def _node_update(cp, s, v, pos, agg_s, agg_v, agg_p):
    u1 = cp["u1"]
    u2 = cp["u2"]
    u = jnp.concatenate([s, agg_s], axis=-1) @ u1["W"] + u1["b"]
    u = u * jax.nn.sigmoid(u)
    uo = u @ u2["W"] + u2["b"]
    s2 = s + uo[:, :SDIM]
    v2 = v + uo[:, SDIM:][:, None, :] * agg_v
    return s2, v2, pos + agg_p


def _lnorm_(np_, s, v):
    mu = jnp.mean(s, axis=-1, keepdims=True)
    var = jnp.var(s, axis=-1, keepdims=True)
    s2 = (s - mu) / jnp.sqrt(var + 1e-5) * np_["gamma"] + np_["beta"]
    vn = jnp.sqrt(jnp.mean(jnp.sum(v * v, axis=1), axis=-1) + 1e-6)
    return s2, v / vn[:, None, None]


def _vpad(v):
    out = []
    for c in range(3):
        out.append(jnp.zeros((N, 128), _F32).at[:, :VDIM].set(v[:, c, :]))
    return out


def _radius_graph_host(pos, batch):
    pz = pos
    d2 = jnp.sum((pz[:, None, :] - pz[None, :, :]) ** 2, axis=-1)
    same = (batch[:, None] == batch[None, :]) & (~jnp.eye(N, dtype=bool))
    d2m = jnp.where(same, d2, 1e10)
    negv, idx = jax.lax.top_k(-d2m, K)
    valid = ((-negv) < CUT * CUT).astype(_F32)
    return idx, valid


def kernel(s, v, p, edge_index_global, edge_attr_global, batch, params):
    src_g = edge_index_global[0]
    dst_g = edge_index_global[1]
    E_dense = jnp.zeros((N, N, EDIM), _F32).at[src_g, dst_g].set(edge_attr_global)

    src2d_g = src_g.astype(jnp.int32).reshape(EG, 1)
    dst2d_g = dst_g.astype(jnp.int32).reshape(EG, 1)

    def pp_layer(cp, s, v, pos):
        W1a, W1b, b1, wpack = _prep_conv_weights(cp)
        A = s @ W1a + b1
        B = s @ W1b
        pn = pos / jnp.linalg.norm(pos, axis=1, keepdims=True)
        outs, outv, outp, outc = _pp_conv_aggregate(
            src2d_g, dst2d_g, edge_attr_global, A, B, pos, pn, wpack)
        cnt = outc + 1e-6
        agg_v = (outv / cnt).reshape(N, 3, 128)[:, :, :VDIM]
        return _node_update(cp, s, v, pos, outs, agg_v, outp / cnt)

    def mid_layer(cp, s, v, pos, idx, valid):
        W1a, W1b, b1, wpack = _prep_conv_weights(cp)
        A = s @ W1a + b1
        B = s @ W1b
        pn = pos / jnp.linalg.norm(pos, axis=1, keepdims=True)
        src2d = idx.astype(jnp.int32).reshape(N * K, 1)
        valid2d = valid.reshape(N * K, 1)
        dst_flat = jnp.repeat(jnp.arange(N), K)
        eattr = E_dense[src2d[:, 0], dst_flat]
        outs, outv, outp = _mid_conv_aggregate(
            src2d, valid2d, eattr, A, B, pos, pn, _vpad(v), wpack)
        agg_v = outv.reshape(N, 3, 128)[:, :, :VDIM]
        return _node_update(cp, s, v, pos, outs, agg_v, outp)

    s, v, p = pp_layer(params["pre"], s, v, p)
    for i in range(NCONV):
        idx, valid = _radius_graph_host(p, batch)
        s, v = _lnorm_(params["norms"][i], s, v)
        s, v, p = mid_layer(params["convs"][i], s, v, p, idx, valid)
    s, v = _lnorm_(params["pn0"], s, v)
    s, v, p = pp_layer(params["post"], s, v, p)
    s, v = _lnorm_(params["pn1"], s, v)

    e = s @ params["ep1"]["W"] + params["ep1"]["b"]
    e = e * jax.nn.sigmoid(e)
    e = e @ params["ep2"]["W"] + params["ep2"]["b"]
    e = e[src_g] + e[dst_g]
    e = edge_attr_global + e
    eh = e @ params["eq1"]["W"] + params["eq1"]["b"]
    eh = eh * jax.nn.sigmoid(eh)
    e = eh @ params["eq2"]["W"] + params["eq2"]["b"]
    return s, v, e, p
